# baseline (speedup 1.0000x reference)
import math

import jax
import jax.numpy as jnp
from jax import lax
from jax.experimental import pallas as pl
from jax.experimental.pallas import tpu as pltpu
from jax.experimental.pallas import tpu_sc as plsc

D_MODEL = 64
SCALE = math.sqrt(D_MODEL)

_NC = 2
_NS = 16
_NW = _NC * _NS

_CHUNK = 256
_IDX_PER_STREAM = 128


def _make_embed(B: int):
    assert B % (_NW * _CHUNK * 2) == 0, B
    bpw = B // _NW
    nchunk = bpw // _CHUNK
    ngath = _CHUNK // _IDX_PER_STREAM

    mesh = plsc.VectorSubcoreMesh(core_axis_name="c", subcore_axis_name="s")

    def body(table_hbm, idx_hbm, out_hbm, idx_v, rows0, rows1, st0, st1,
             gsem0, gsem1, ssem0, ssem1):
        wid = lax.axis_index("s") * _NC + lax.axis_index("c")
        base = wid * bpw

        pltpu.sync_copy(idx_hbm.at[pl.ds(base, bpw)], idx_v)

        def fire_gather(g, rows, gsem):
            for j in range(ngath):
                pltpu.async_copy(
                    table_hbm.at[idx_v.at[pl.ds(g * _CHUNK
                                                + j * _IDX_PER_STREAM,
                                                _IDX_PER_STREAM)]],
                    rows.at[pl.ds(j * _IDX_PER_STREAM, _IDX_PER_STREAM)],
                    gsem,
                )

        def wait_gather(rows, gsem):
            pltpu.make_async_copy(out_hbm.at[pl.ds(0, _CHUNK)], rows,
                                  gsem).wait()

        def fire_store(g, st, ssem):
            pltpu.async_copy(st, out_hbm.at[pl.ds(base + g * _CHUNK, _CHUNK)],
                             ssem)

        def wait_store(st, ssem):
            pltpu.make_async_copy(st, out_hbm.at[pl.ds(0, _CHUNK)],
                                  ssem).wait()

        def scale(rows, st):
            @pl.loop(0, _CHUNK, unroll=8)
            def _scale(r):
                for j in range(D_MODEL // 16):
                    sl = pl.ds(j * 16, 16)
                    st[r, sl] = rows[r, sl] * SCALE

        def step(g, rows, st, gsem, ssem, first=False, last=False):
            wait_gather(rows, gsem)
            if not first:
                wait_store(st, ssem)
            scale(rows, st)
            fire_store(g, st, ssem)
            if not last:
                fire_gather(g + 2, rows, gsem)

        bufs = ((rows0, st0, gsem0, ssem0), (rows1, st1, gsem1, ssem1))

        fire_gather(0, rows0, gsem0)
        fire_gather(1, rows1, gsem1)
        step(0, *bufs[0], first=True)
        step(1, *bufs[1], first=True)

        @pl.loop(2, nchunk - 2, step=2)
        def _steady(g0):
            step(g0, *bufs[0])
            step(g0 + 1, *bufs[1])

        step(nchunk - 2, *bufs[0], last=True)
        step(nchunk - 1, *bufs[1], last=True)
        wait_store(st0, ssem0)
        wait_store(st1, ssem1)

    return pl.kernel(
        body,
        out_type=jax.ShapeDtypeStruct((B, D_MODEL), jnp.float32),
        mesh=mesh,
        scratch_types=[
            pltpu.VMEM((B // _NW,), jnp.int32),
            pltpu.VMEM((_CHUNK, D_MODEL), jnp.float32),
            pltpu.VMEM((_CHUNK, D_MODEL), jnp.float32),
            pltpu.VMEM((_CHUNK, D_MODEL), jnp.float32),
            pltpu.VMEM((_CHUNK, D_MODEL), jnp.float32),
            pltpu.SemaphoreType.DMA,
            pltpu.SemaphoreType.DMA,
            pltpu.SemaphoreType.DMA,
            pltpu.SemaphoreType.DMA,
        ],
        compiler_params=pltpu.CompilerParams(use_tc_tiling_on_sc=False),
    )


def kernel(x, table):
    batch, hist = x.shape
    idx = x.reshape(-1).astype(jnp.int32)
    out = _make_embed(idx.shape[0])(table, idx)
    return out.reshape(batch, hist, D_MODEL)
